# Initial kernel scaffold; baseline (speedup 1.0000x reference)
import jax
import jax.numpy as jnp
from jax.experimental import pallas as pl


def kernel(info_bits, r, emb_obs, emb_lab, W_c1, b_c1, W_c2, b_c2, W_b1, b_b1, W_b2, b_b2, W_llr, b_llr):
    raise NotImplementedError("write your pallas kernel here")


# speedup vs baseline: 1.6242x; 1.6242x over previous
import numpy as np
import jax
import jax.numpy as jnp
from jax.experimental import pallas as pl

_B = 256
_N = 512
_K = 256
_D = 16
_H = 32
_THRESH = 0.25
_perm = np.random.RandomState(0).permutation(_N)
_INFO_SET = np.sort(_perm[:_K]).astype(np.int32)
_POS2COL = {int(p): c for c, p in enumerate(_INFO_SET)}
_BT = 64


def _gen_matrix(n):
    if n == 1:
        return np.ones((1, 1), dtype=np.float32)
    m = n // 2
    g = _gen_matrix(m)
    G = np.zeros((n, n), dtype=np.float32)
    G[:m, 0::2] = g
    G[m:, 0::2] = g
    G[m:, 1::2] = g
    return G


_G = _gen_matrix(_N)


def _encoder_body(info_ref, r_ref, emb_obs_ref, emb_lab_ref,
                  Wc1_ref, bc1_ref, Wc2_ref, bc2_ref,
                  Wb1_ref, bb1_ref, Wb2_ref, bb2_ref,
                  Wllr_ref, bllr_ref, g_ref,
                  x_ref, u_ref, p0_ref, p1_ref):
    W_c1 = Wc1_ref[...]
    b_c1 = bc1_ref[...]
    W_c2 = Wc2_ref[...]
    b_c2 = bc2_ref[...]
    W_b1 = Wb1_ref[...]
    b_b1 = bb1_ref[...]
    W_b2 = Wb2_ref[...]
    b_b2 = bb2_ref[...]
    W_llr = Wllr_ref[...]
    b_llr = bllr_ref[...]
    emb0 = emb_lab_ref[0:1, :]
    emb1 = emb_lab_ref[1:2, :]
    rfull = r_ref[...]
    ifull = info_ref[...]

    def checknode(t2):
        h = jnp.maximum(jnp.dot(t2, W_c1) + b_c1, 0.0)
        return jnp.dot(h, W_c2) + b_c2

    def bitnode(t3):
        h = jnp.maximum(jnp.dot(t3, W_b1) + b_b1, 0.0)
        return jnp.dot(h, W_b2) + b_b2

    def leaf(ef, base):
        z = jnp.dot(ef, W_llr) + b_llr
        m = jnp.max(z, axis=-1, keepdims=True)
        ez = jnp.exp(z - m)
        s = jnp.sum(ez, axis=-1, keepdims=True)
        p = ez / s
        p0 = p[:, 0:1]
        p1 = p[:, 1:2]
        rr = rfull[:, base:base + 1]
        hd = (rr > p0).astype(jnp.int32)
        col = _POS2COL.get(base)
        if col is None:
            x = hd
        else:
            fv = ifull[:, col:col + 1]
            cond = jnp.abs(p0 - 0.5) > _THRESH
            x = jnp.where(cond, hd, fv)
        return x, p0, p1

    def encode(ef, base, n):
        if n == 1:
            x, p0, p1 = leaf(ef, base)
            return x, x, p0, p1
        m = n // 2
        hb = m * _BT
        t2 = jnp.concatenate([ef[:hb], ef[hb:]], axis=1)
        u1est = checknode(t2)
        u1, x1, pl0, pl1 = encode(u1est, base, m)
        ue = jnp.where(x1 == 1, emb1, emb0)
        u2est = bitnode(jnp.concatenate([t2, ue], axis=1))
        u2, x2, pr0, pr1 = encode(u2est, base + m, m)
        u = jnp.concatenate([u1, u2], axis=1)
        p0 = jnp.concatenate([pl0, pr0], axis=1)
        p1 = jnp.concatenate([pl1, pr1], axis=1)
        v = jnp.bitwise_xor(x1, x2)
        x = jnp.concatenate([v, x2], axis=0)
        return u, x, p0, p1

    e_root = jnp.broadcast_to(emb_obs_ref[2:3, :], (_N * _BT, _D))
    u, _xbr, p0, p1 = encode(e_root, 0, _N)
    xf = jnp.dot(u.astype(jnp.float32), g_ref[...])
    x_ref[...] = jnp.bitwise_and(xf.astype(jnp.int32), 1)
    u_ref[...] = u
    p0_ref[...] = p0
    p1_ref[...] = p1


def kernel(info_bits, r, emb_obs, emb_lab, W_c1, b_c1, W_c2, b_c2,
           W_b1, b_b1, W_b2, b_b2, W_llr, b_llr):
    def _tile(shape):
        return pl.BlockSpec(shape, lambda i: (0,) * len(shape))

    grid = _B // _BT
    x2d, u2d, p0, p1 = pl.pallas_call(
        _encoder_body,
        grid=(grid,),
        in_specs=[
            pl.BlockSpec((_BT, _K), lambda i: (i, 0)),
            pl.BlockSpec((_BT, _N), lambda i: (i, 0)),
            _tile((3, _D)), _tile((3, _D)),
            _tile((2 * _D, _H)), _tile((_H,)), _tile((_H, _D)), _tile((_D,)),
            _tile((3 * _D, _H)), _tile((_H,)), _tile((_H, _D)), _tile((_D,)),
            _tile((_D, 2)), _tile((2,)),
            _tile((_N, _N)),
        ],
        out_specs=[
            pl.BlockSpec((_BT, _N), lambda i: (i, 0)),
            pl.BlockSpec((_BT, _N), lambda i: (i, 0)),
            pl.BlockSpec((_BT, _N), lambda i: (i, 0)),
            pl.BlockSpec((_BT, _N), lambda i: (i, 0)),
        ],
        out_shape=[
            jax.ShapeDtypeStruct((_B, _N), jnp.int32),
            jax.ShapeDtypeStruct((_B, _N), jnp.int32),
            jax.ShapeDtypeStruct((_B, _N), jnp.float32),
            jax.ShapeDtypeStruct((_B, _N), jnp.float32),
        ],
    )(info_bits, r, emb_obs, emb_lab, W_c1, b_c1, W_c2, b_c2,
      W_b1, b_b1, W_b2, b_b2, W_llr, b_llr, jnp.asarray(_G))

    x = x2d[..., None]
    u = u2d[..., None]
    p_u = jnp.concatenate([p0[..., None], p1[..., None]], axis=2)
    fmask = np.ones((_N, 1), dtype=np.int32)
    fmask[_INFO_SET, 0] = 2
    f = jnp.broadcast_to(jnp.asarray(fmask)[None], (_B, _N, 1))
    return x, f, u, p_u, r


# speedup vs baseline: 2.8742x; 1.7696x over previous
import numpy as np
import jax
import jax.numpy as jnp
from jax.experimental import pallas as pl
from jax.experimental.pallas import tpu as pltpu

_B = 256
_N = 512
_NH = _N // 2
_K = 256
_D = 16
_H = 32
_THRESH = 0.25
_perm = np.random.RandomState(0).permutation(_N)
_INFO_SET = np.sort(_perm[:_K]).astype(np.int32)
_POS2COL = {int(p): c for c, p in enumerate(_INFO_SET)}
_BT = 128
_CHUNK = 8192
_X9CH = max(1, _NH * _BT // _CHUNK)
_X9ROWS = _NH * _BT // _X9CH


def _gen_matrix(n):
    if n == 1:
        return np.ones((1, 1), dtype=np.float32)
    m = n // 2
    g = _gen_matrix(m)
    G = np.zeros((n, n), dtype=np.float32)
    G[:m, 0::2] = g
    G[m:, 0::2] = g
    G[m:, 1::2] = g
    return G


_G = _gen_matrix(_N)


def _norm(chunks):
    total = sum(c.shape[0] for c in chunks)
    if len(chunks) == 1:
        return chunks
    if total <= _CHUNK:
        return [jnp.concatenate(chunks, axis=0)]
    out = chunks
    while out[0].shape[0] < _CHUNK:
        out = [jnp.concatenate([out[i], out[i + 1]], axis=0)
               for i in range(0, len(out), 2)]
    return out


def _common(info_ref, r_ref, emb_obs_ref, emb_lab_ref,
            Wc1_ref, bc1_ref, Wc2_ref, bc2_ref,
            Wb1_ref, bb1_ref, Wb2_ref, bb2_ref,
            Wllr_ref, bllr_ref):
    W_c1 = Wc1_ref[...]
    b_c1 = bc1_ref[...]
    W_c2 = Wc2_ref[...]
    b_c2 = bc2_ref[...]
    W_b1 = Wb1_ref[...]
    b_b1 = bb1_ref[...]
    W_b2 = Wb2_ref[...]
    b_b2 = bb2_ref[...]
    W_llr = Wllr_ref[...]
    b_llr = bllr_ref[...]
    emb0 = emb_lab_ref[0:1, :]
    emb1 = emb_lab_ref[1:2, :]
    rfull = r_ref[...]
    ifull = info_ref[...]

    def checknode(t2):
        h = jnp.maximum(jnp.dot(t2, W_c1) + b_c1, 0.0)
        return jnp.dot(h, W_c2) + b_c2

    def bitnode(t3):
        h = jnp.maximum(jnp.dot(t3, W_b1) + b_b1, 0.0)
        return jnp.dot(h, W_b2) + b_b2

    def ue_of(x1c):
        return jnp.where(x1c == 1, emb1, emb0)

    def leaf(ef, base, const):
        z = jnp.dot(ef, W_llr) + b_llr
        m = jnp.max(z, axis=-1, keepdims=True)
        ez = jnp.exp(z - m)
        s = jnp.sum(ez, axis=-1, keepdims=True)
        p = ez / s
        p0 = p[:, 0:1]
        p1 = p[:, 1:2]
        rr = rfull[:, base:base + 1]
        hd = (rr > p0).astype(jnp.int32)
        col = _POS2COL.get(base)
        if col is None:
            x = hd
        else:
            fv = ifull[:, col:col + 1]
            cond = jnp.abs(p0 - 0.5) > _THRESH
            x = jnp.where(cond, hd, fv)
        if const:
            p0 = jnp.broadcast_to(p0, (_BT, 1))
            p1 = jnp.broadcast_to(p1, (_BT, 1))
        return x, p0, p1

    def encode(ef, base, n, const, need_x=True):
        if n == 1:
            x, p0, p1 = leaf(ef if const else ef[0], base, const)
            return x, [x], p0, p1
        m = n // 2
        if const:
            t2 = jnp.concatenate([ef, ef], axis=1)
            u1est = checknode(t2)
        else:
            nc = len(ef)
            if nc == 1:
                half = ef[0].shape[0] // 2
                t2ch = [jnp.concatenate([ef[0][:half], ef[0][half:]], axis=1)]
            else:
                t2ch = [jnp.concatenate([ef[j], ef[j + nc // 2]], axis=1)
                        for j in range(nc // 2)]
            u1est = [checknode(t) for t in t2ch]

        u1, x1ch, pl0, pl1 = encode(u1est, base, m, const)

        u2est = []
        if const:
            for x1c in x1ch:
                t2full = jnp.broadcast_to(t2, (x1c.shape[0], 2 * _D))
                u2est.append(bitnode(
                    jnp.concatenate([t2full, ue_of(x1c)], axis=1)))
        else:
            for j, t2c in enumerate(t2ch):
                x1c = x1ch[j] if len(x1ch) > 1 else x1ch[0]
                u2est.append(bitnode(
                    jnp.concatenate([t2c, ue_of(x1c)], axis=1)))
        u2est = _norm(u2est)

        u2, x2ch, pr0, pr1 = encode(u2est, base + m, m, False)

        u = jnp.concatenate([u1, u2], axis=1)
        p0 = jnp.concatenate([pl0, pr0], axis=1)
        p1 = jnp.concatenate([pl1, pr1], axis=1)
        if not need_x:
            return u, None, p0, p1
        vch = [jnp.bitwise_xor(a, b) for a, b in zip(x1ch, x2ch)]
        xch = _norm(vch + x2ch)
        return u, xch, p0, p1

    def root_t2():
        e_row = emb_obs_ref[2:3, :]
        return jnp.concatenate([e_row, e_row], axis=1)

    return checknode, bitnode, ue_of, encode, root_t2


def _left_body(info_ref, r_ref, emb_obs_ref, emb_lab_ref,
               Wc1_ref, bc1_ref, Wc2_ref, bc2_ref,
               Wb1_ref, bb1_ref, Wb2_ref, bb2_ref,
               Wllr_ref, bllr_ref,
               uL_ref, p0L_ref, p1L_ref, x9_ref):
    checknode, _bitnode, _ue, encode, root_t2 = _common(
        info_ref, r_ref, emb_obs_ref, emb_lab_ref,
        Wc1_ref, bc1_ref, Wc2_ref, bc2_ref,
        Wb1_ref, bb1_ref, Wb2_ref, bb2_ref, Wllr_ref, bllr_ref)
    u1est = checknode(root_t2())
    u, xch, p0, p1 = encode(u1est, 0, _NH, True)
    uL_ref[...] = u
    p0L_ref[...] = p0
    p1L_ref[...] = p1
    for j, c in enumerate(_norm(xch)):
        x9_ref[0, 0:c.shape[0], j:j + 1] = c


def _right_body(info_ref, r_ref, emb_obs_ref, emb_lab_ref,
                Wc1_ref, bc1_ref, Wc2_ref, bc2_ref,
                Wb1_ref, bb1_ref, Wb2_ref, bb2_ref,
                Wllr_ref, bllr_ref, g_ref, x9_ref, uL_ref,
                x_ref, uR_ref, p0R_ref, p1R_ref):
    _checknode, bitnode, ue_of, encode, root_t2 = _common(
        info_ref, r_ref, emb_obs_ref, emb_lab_ref,
        Wc1_ref, bc1_ref, Wc2_ref, bc2_ref,
        Wb1_ref, bb1_ref, Wb2_ref, bb2_ref, Wllr_ref, bllr_ref)
    t2 = root_t2()
    u2est = []
    for j in range(_X9CH):
        x1c = x9_ref[0, 0:_X9ROWS, j:j + 1]
        t2full = jnp.broadcast_to(t2, (_X9ROWS, 2 * _D))
        u2est.append(bitnode(
            jnp.concatenate([t2full, ue_of(x1c)], axis=1)))
    u2, _x, p0, p1 = encode(_norm(u2est), _NH, _NH, False, need_x=False)
    uR_ref[...] = u2
    p0R_ref[...] = p0
    p1R_ref[...] = p1
    u = jnp.concatenate([uL_ref[...], u2], axis=1)
    xf = jnp.dot(u.astype(jnp.float32), g_ref[...])
    x_ref[...] = jnp.bitwise_and(xf.astype(jnp.int32), 1)


def _tile(shape):
    return pl.BlockSpec(shape, lambda i: (0,) * len(shape))


_W_SPECS = [
    _tile((3, _D)), _tile((3, _D)),
    _tile((2 * _D, _H)), _tile((_H,)), _tile((_H, _D)), _tile((_D,)),
    _tile((3 * _D, _H)), _tile((_H,)), _tile((_H, _D)), _tile((_D,)),
    _tile((_D, 2)), _tile((2,)),
]


def kernel(info_bits, r, emb_obs, emb_lab, W_c1, b_c1, W_c2, b_c2,
           W_b1, b_b1, W_b2, b_b2, W_llr, b_llr):
    grid = _B // _BT
    weights = (emb_obs, emb_lab, W_c1, b_c1, W_c2, b_c2,
               W_b1, b_b1, W_b2, b_b2, W_llr, b_llr)

    uL, p0L, p1L, x9 = pl.pallas_call(
        _left_body,
        grid=(grid,),
        compiler_params=pltpu.CompilerParams(
            dimension_semantics=("parallel",),
        ),
        in_specs=[
            pl.BlockSpec((_BT, _K), lambda i: (i, 0)),
            pl.BlockSpec((_BT, _N), lambda i: (i, 0)),
        ] + _W_SPECS,
        out_specs=[
            pl.BlockSpec((_BT, _NH), lambda i: (i, 0)),
            pl.BlockSpec((_BT, _NH), lambda i: (i, 0)),
            pl.BlockSpec((_BT, _NH), lambda i: (i, 0)),
            pl.BlockSpec((1, _X9ROWS, _X9CH), lambda i: (i, 0, 0)),
        ],
        out_shape=[
            jax.ShapeDtypeStruct((_B, _NH), jnp.int32),
            jax.ShapeDtypeStruct((_B, _NH), jnp.float32),
            jax.ShapeDtypeStruct((_B, _NH), jnp.float32),
            jax.ShapeDtypeStruct((grid, _X9ROWS, _X9CH), jnp.int32),
        ],
    )(info_bits, r, *weights)

    x2d, uR, p0R, p1R = pl.pallas_call(
        _right_body,
        grid=(grid,),
        compiler_params=pltpu.CompilerParams(
            dimension_semantics=("parallel",),
        ),
        in_specs=[
            pl.BlockSpec((_BT, _K), lambda i: (i, 0)),
            pl.BlockSpec((_BT, _N), lambda i: (i, 0)),
        ] + _W_SPECS + [
            _tile((_N, _N)),
            pl.BlockSpec((1, _X9ROWS, _X9CH), lambda i: (i, 0, 0)),
            pl.BlockSpec((_BT, _NH), lambda i: (i, 0)),
        ],
        out_specs=[
            pl.BlockSpec((_BT, _N), lambda i: (i, 0)),
            pl.BlockSpec((_BT, _NH), lambda i: (i, 0)),
            pl.BlockSpec((_BT, _NH), lambda i: (i, 0)),
            pl.BlockSpec((_BT, _NH), lambda i: (i, 0)),
        ],
        out_shape=[
            jax.ShapeDtypeStruct((_B, _N), jnp.int32),
            jax.ShapeDtypeStruct((_B, _NH), jnp.int32),
            jax.ShapeDtypeStruct((_B, _NH), jnp.float32),
            jax.ShapeDtypeStruct((_B, _NH), jnp.float32),
        ],
    )(info_bits, r, *weights, jnp.asarray(_G), x9, uL)

    x = x2d[..., None]
    u = jnp.concatenate([uL, uR], axis=1)[..., None]
    p0 = jnp.concatenate([p0L, p0R], axis=1)
    p1 = jnp.concatenate([p1L, p1R], axis=1)
    p_u = jnp.concatenate([p0[..., None], p1[..., None]], axis=2)
    fmask = np.ones((_N, 1), dtype=np.int32)
    fmask[_INFO_SET, 0] = 2
    f = jnp.broadcast_to(jnp.asarray(fmask)[None], (_B, _N, 1))
    return x, f, u, p_u, r


# speedup vs baseline: 4.9579x; 1.7249x over previous
import numpy as np
import jax
import jax.numpy as jnp
from jax.experimental import pallas as pl
from jax.experimental.pallas import tpu as pltpu

_B = 256
_N = 512
_NQ = _N // 4
_K = 256
_D = 16
_H = 32
_THRESH = 0.25
_perm = np.random.RandomState(0).permutation(_N)
_INFO_SET = np.sort(_perm[:_K]).astype(np.int32)
_POS2COL = {int(p): c for c, p in enumerate(_INFO_SET)}
_BT = 256
_CHUNK = 8192
_QCH = _NQ * _BT // _CHUNK
_HCH = 2 * _QCH


def _gen_matrix(n):
    if n == 1:
        return np.ones((1, 1), dtype=np.float32)
    m = n // 2
    g = _gen_matrix(m)
    G = np.zeros((n, n), dtype=np.float32)
    G[:m, 0::2] = g
    G[m:, 0::2] = g
    G[m:, 1::2] = g
    return G


_G = _gen_matrix(_N)


def _norm(chunks):
    total = sum(c.shape[0] for c in chunks)
    if len(chunks) == 1:
        return chunks
    if total <= _CHUNK:
        return [jnp.concatenate(chunks, axis=0)]
    out = chunks
    while out[0].shape[0] < _CHUNK:
        out = [jnp.concatenate([out[i], out[i + 1]], axis=0)
               for i in range(0, len(out), 2)]
    return out


def _common(info_ref, r_ref, emb_obs_ref, emb_lab_ref,
            Wc1_ref, bc1_ref, Wc2_ref, bc2_ref,
            Wb1_ref, bb1_ref, Wb2_ref, bb2_ref,
            Wllr_ref, bllr_ref):
    W_c1 = Wc1_ref[...]
    b_c1 = bc1_ref[...]
    W_c2 = Wc2_ref[...]
    b_c2 = bc2_ref[...]
    W_b1 = Wb1_ref[...]
    b_b1 = bb1_ref[...]
    W_b2 = Wb2_ref[...]
    b_b2 = bb2_ref[...]
    W_llr = Wllr_ref[...]
    b_llr = bllr_ref[...]
    emb0 = emb_lab_ref[0:1, :]
    emb1 = emb_lab_ref[1:2, :]
    rfull = r_ref[...]
    ifull = info_ref[...]

    def checknode(t2):
        h = jnp.maximum(jnp.dot(t2, W_c1) + b_c1, 0.0)
        return jnp.dot(h, W_c2) + b_c2

    def bitnode(t3):
        h = jnp.maximum(jnp.dot(t3, W_b1) + b_b1, 0.0)
        return jnp.dot(h, W_b2) + b_b2

    def ue_of(x1c):
        return jnp.where(x1c == 1, emb1, emb0)

    def leaf(ef, base, const):
        z = jnp.dot(ef, W_llr) + b_llr
        m = jnp.max(z, axis=-1, keepdims=True)
        ez = jnp.exp(z - m)
        s = jnp.sum(ez, axis=-1, keepdims=True)
        p = ez / s
        p0 = p[:, 0:1]
        p1 = p[:, 1:2]
        rr = rfull[:, base:base + 1]
        hd = (rr > p0).astype(jnp.int32)
        col = _POS2COL.get(base)
        if col is None:
            x = hd
        else:
            fv = ifull[:, col:col + 1]
            cond = jnp.abs(p0 - 0.5) > _THRESH
            x = jnp.where(cond, hd, fv)
        if const:
            p0 = jnp.broadcast_to(p0, (_BT, 1))
            p1 = jnp.broadcast_to(p1, (_BT, 1))
        return x, p0, p1

    def encode(ef, base, n, const, need_x=True):
        if n == 1:
            x, p0, p1 = leaf(ef if const else ef[0], base, const)
            return x, [x], p0, p1
        m = n // 2
        if const:
            t2 = jnp.concatenate([ef, ef], axis=1)
            u1est = checknode(t2)
        else:
            nc = len(ef)
            if nc == 1:
                half = ef[0].shape[0] // 2
                t2ch = [jnp.concatenate([ef[0][:half], ef[0][half:]], axis=1)]
            else:
                t2ch = [jnp.concatenate([ef[j], ef[j + nc // 2]], axis=1)
                        for j in range(nc // 2)]
            u1est = [checknode(t) for t in t2ch]

        u1, x1ch, pl0, pl1 = encode(u1est, base, m, const)

        u2est = []
        if const:
            for x1c in x1ch:
                t2full = jnp.broadcast_to(t2, (x1c.shape[0], 2 * _D))
                u2est.append(bitnode(
                    jnp.concatenate([t2full, ue_of(x1c)], axis=1)))
        else:
            for j, t2c in enumerate(t2ch):
                x1c = x1ch[j] if len(x1ch) > 1 else x1ch[0]
                u2est.append(bitnode(
                    jnp.concatenate([t2c, ue_of(x1c)], axis=1)))
        u2est = _norm(u2est)

        u2, x2ch, pr0, pr1 = encode(u2est, base + m, m, False)

        u = jnp.concatenate([u1, u2], axis=1)
        p0 = jnp.concatenate([pl0, pr0], axis=1)
        p1 = jnp.concatenate([pl1, pr1], axis=1)
        if not need_x:
            return u, None, p0, p1
        vch = [jnp.bitwise_xor(a, b) for a, b in zip(x1ch, x2ch)]
        xch = _norm(vch + x2ch)
        return u, xch, p0, p1

    def spine():
        e_row = emb_obs_ref[2:3, :]
        t2_root = jnp.concatenate([e_row, e_row], axis=1)
        ef_8l = checknode(t2_root)
        t2_8l = jnp.concatenate([ef_8l, ef_8l], axis=1)
        ef_q1 = checknode(t2_8l)
        return t2_root, t2_8l, ef_q1

    return checknode, bitnode, ue_of, encode, spine


_STD = 14


def _q1_body(*refs):
    (uq_ref, p0_ref, p1_ref, xb_ref) = refs[_STD:]
    _ck, _bn, _ue, encode, spine = _common(*refs[:_STD])
    _t2r, _t28, ef_q1 = spine()
    u, xch, p0, p1 = encode(ef_q1, 0, _NQ, True)
    uq_ref[...] = u
    p0_ref[...] = p0
    p1_ref[...] = p1
    for j, c in enumerate(_norm(xch)):
        xb_ref[0, 0:c.shape[0], j:j + 1] = c


def _q2_body(*refs):
    xq1_ref = refs[_STD]
    (uq_ref, p0_ref, p1_ref, xb_ref) = refs[_STD + 1:]
    _ck, bitnode, ue_of, encode, spine = _common(*refs[:_STD])
    _t2r, t2_8l, _efq1 = spine()
    u2est = []
    for j in range(_QCH):
        x1c = xq1_ref[0, :, j:j + 1]
        t2full = jnp.broadcast_to(t2_8l, (_CHUNK, 2 * _D))
        u2est.append(bitnode(jnp.concatenate([t2full, ue_of(x1c)], axis=1)))
    u, x2ch, p0, p1 = encode(_norm(u2est), _NQ, _NQ, False)
    uq_ref[...] = u
    p0_ref[...] = p0
    p1_ref[...] = p1
    for j in range(_QCH):
        v = jnp.bitwise_xor(xq1_ref[0, :, j:j + 1], x2ch[j])
        xb_ref[0, :, j:j + 1] = v
    for j in range(_QCH):
        xb_ref[0, :, _QCH + j: _QCH + j + 1] = x2ch[j]


def _q3_body(*refs):
    x8l_ref = refs[_STD]
    (uq_ref, p0_ref, p1_ref, xb_ref, t2b_ref) = refs[_STD + 1:]
    checknode, bitnode, ue_of, encode, spine = _common(*refs[:_STD])
    t2_root, _t28, _efq1 = spine()
    ef8r = []
    for j in range(_HCH):
        x1c = x8l_ref[0, :, j:j + 1]
        t2full = jnp.broadcast_to(t2_root, (_CHUNK, 2 * _D))
        ef8r.append(bitnode(jnp.concatenate([t2full, ue_of(x1c)], axis=1)))
    t2ch = [jnp.concatenate([ef8r[j], ef8r[j + _HCH // 2]], axis=1)
            for j in range(_HCH // 2)]
    for j, t in enumerate(t2ch):
        t2b_ref[0, :, 32 * j: 32 * (j + 1)] = t
    u1est = [checknode(t) for t in t2ch]
    u, xch, p0, p1 = encode(u1est, 2 * _NQ, _NQ, False)
    uq_ref[...] = u
    p0_ref[...] = p0
    p1_ref[...] = p1
    for j, c in enumerate(_norm(xch)):
        xb_ref[0, 0:c.shape[0], j:j + 1] = c


def _q4_body(*refs):
    g_ref, t2b_ref, xq3_ref, u1_ref, u2_ref, u3_ref = refs[_STD:_STD + 6]
    (x_ref, uq_ref, p0_ref, p1_ref) = refs[_STD + 6:]
    _ck, bitnode, ue_of, encode, _spine = _common(*refs[:_STD])
    u2est = []
    for j in range(_QCH):
        t2c = t2b_ref[0, :, 32 * j: 32 * (j + 1)]
        x1c = xq3_ref[0, :, j:j + 1]
        u2est.append(bitnode(jnp.concatenate([t2c, ue_of(x1c)], axis=1)))
    u, _x, p0, p1 = encode(_norm(u2est), 3 * _NQ, _NQ, False, need_x=False)
    uq_ref[...] = u
    p0_ref[...] = p0
    p1_ref[...] = p1
    ufull = jnp.concatenate([u1_ref[...], u2_ref[...], u3_ref[...], u],
                            axis=1)
    xf = jnp.dot(ufull.astype(jnp.float32), g_ref[...])
    x_ref[...] = jnp.bitwise_and(xf.astype(jnp.int32), 1)


def _full(shape):
    return pl.BlockSpec(shape, lambda: (0,) * len(shape))


_STD_SPECS = [
    _full((_BT, _K)), _full((_BT, _N)),
    _full((3, _D)), _full((3, _D)),
    _full((2 * _D, _H)), _full((_H,)), _full((_H, _D)), _full((_D,)),
    _full((3 * _D, _H)), _full((_H,)), _full((_H, _D)), _full((_D,)),
    _full((_D, 2)), _full((2,)),
]

_QOUT = [
    jax.ShapeDtypeStruct((_BT, _NQ), jnp.int32),
    jax.ShapeDtypeStruct((_BT, _NQ), jnp.float32),
    jax.ShapeDtypeStruct((_BT, _NQ), jnp.float32),
]
_QOUT_SPECS = [_full((_BT, _NQ)), _full((_BT, _NQ)), _full((_BT, _NQ))]


def _bits(nch):
    return (jax.ShapeDtypeStruct((1, _CHUNK, nch), jnp.int32),
            _full((1, _CHUNK, nch)))


def kernel(info_bits, r, emb_obs, emb_lab, W_c1, b_c1, W_c2, b_c2,
           W_b1, b_b1, W_b2, b_b2, W_llr, b_llr):
    std = (info_bits, r, emb_obs, emb_lab, W_c1, b_c1, W_c2, b_c2,
           W_b1, b_b1, W_b2, b_b2, W_llr, b_llr)
    xq1_t, xq1_s = _bits(_QCH)
    x8l_t, x8l_s = _bits(_HCH)
    xq3_t, xq3_s = _bits(_QCH)
    t2b_t = jax.ShapeDtypeStruct((1, _CHUNK, 32 * _QCH), jnp.float32)
    t2b_s = _full((1, _CHUNK, 32 * _QCH))

    u1, p01, p11, xq1 = pl.pallas_call(
        _q1_body, in_specs=list(_STD_SPECS),
        out_specs=_QOUT_SPECS + [xq1_s],
        out_shape=_QOUT + [xq1_t],
    )(*std)

    u2, p02, p12, x8l = pl.pallas_call(
        _q2_body, in_specs=list(_STD_SPECS) + [xq1_s],
        out_specs=_QOUT_SPECS + [x8l_s],
        out_shape=_QOUT + [x8l_t],
    )(*std, xq1)

    u3, p03, p13, xq3, t2b = pl.pallas_call(
        _q3_body, in_specs=list(_STD_SPECS) + [x8l_s],
        out_specs=_QOUT_SPECS + [xq3_s, t2b_s],
        out_shape=_QOUT + [xq3_t, t2b_t],
    )(*std, x8l)

    x2d, u4, p04, p14 = pl.pallas_call(
        _q4_body,
        in_specs=list(_STD_SPECS) + [_full((_N, _N)), t2b_s, xq3_s,
                                     _full((_BT, _NQ)), _full((_BT, _NQ)),
                                     _full((_BT, _NQ))],
        out_specs=[_full((_BT, _N))] + _QOUT_SPECS,
        out_shape=[jax.ShapeDtypeStruct((_BT, _N), jnp.int32)] + _QOUT,
    )(*std, jnp.asarray(_G), t2b, xq3, u1, u2, u3)

    x = x2d[..., None]
    u = jnp.concatenate([u1, u2, u3, u4], axis=1)[..., None]
    p0 = jnp.concatenate([p01, p02, p03, p04], axis=1)
    p1 = jnp.concatenate([p11, p12, p13, p14], axis=1)
    p_u = jnp.concatenate([p0[..., None], p1[..., None]], axis=2)
    fmask = np.ones((_N, 1), dtype=np.int32)
    fmask[_INFO_SET, 0] = 2
    f = jnp.broadcast_to(jnp.asarray(fmask)[None], (_B, _N, 1))
    return x, f, u, p_u, r


# speedup vs baseline: 5.7807x; 1.1660x over previous
import numpy as np
import jax
import jax.numpy as jnp
from jax.experimental import pallas as pl
from jax.experimental.pallas import tpu as pltpu

_B = 256
_N = 512
_NQ = _N // 4
_K = 256
_D = 16
_H = 32
_THRESH = 0.25
_perm = np.random.RandomState(0).permutation(_N)
_INFO_SET = np.sort(_perm[:_K]).astype(np.int32)
_POS2COL = {int(p): c for c, p in enumerate(_INFO_SET)}
_BT = 256
_CHUNK = 8192
_SPEC_ROWS = 8192
_QCH = _NQ * _BT // _CHUNK
_HCH = 2 * _QCH


def _gen_matrix(n):
    if n == 1:
        return np.ones((1, 1), dtype=np.float32)
    m = n // 2
    g = _gen_matrix(m)
    G = np.zeros((n, n), dtype=np.float32)
    G[:m, 0::2] = g
    G[m:, 0::2] = g
    G[m:, 1::2] = g
    return G


_G = _gen_matrix(_N)


def _norm(chunks):
    total = sum(c.shape[0] for c in chunks)
    if len(chunks) == 1:
        return chunks
    if total <= _CHUNK:
        return [jnp.concatenate(chunks, axis=0)]
    out = chunks
    while out[0].shape[0] < _CHUNK:
        out = [jnp.concatenate([out[i], out[i + 1]], axis=0)
               for i in range(0, len(out), 2)]
    return out


def _common(info_ref, r_ref, emb_obs_ref, emb_lab_ref,
            Wc1_ref, bc1_ref, Wc2_ref, bc2_ref,
            Wb1_ref, bb1_ref, Wb2_ref, bb2_ref,
            Wllr_ref, bllr_ref):
    W_c1 = Wc1_ref[...]
    b_c1 = bc1_ref[...]
    W_c2 = Wc2_ref[...]
    b_c2 = bc2_ref[...]
    W_b1 = Wb1_ref[...]
    b_b1 = bb1_ref[...]
    W_b2 = Wb2_ref[...]
    b_b2 = bb2_ref[...]
    W_llr = Wllr_ref[...]
    b_llr = bllr_ref[...]
    emb0 = emb_lab_ref[0:1, :]
    emb1 = emb_lab_ref[1:2, :]
    rfull = r_ref[...]
    ifull = info_ref[...]

    def checknode(t2):
        h = jnp.maximum(jnp.dot(t2, W_c1) + b_c1, 0.0)
        return jnp.dot(h, W_c2) + b_c2

    def bitnode(t3):
        h = jnp.maximum(jnp.dot(t3, W_b1) + b_b1, 0.0)
        return jnp.dot(h, W_b2) + b_b2

    def ue_of(x1c):
        return jnp.where(x1c == 1, emb1, emb0)

    def bit_both(t2row):
        return (bitnode(jnp.concatenate([t2row, emb0], axis=1)),
                bitnode(jnp.concatenate([t2row, emb1], axis=1)))

    def leaf(ef, base, const):
        z = jnp.dot(ef, W_llr) + b_llr
        m = jnp.max(z, axis=-1, keepdims=True)
        ez = jnp.exp(z - m)
        s = jnp.sum(ez, axis=-1, keepdims=True)
        p = ez / s
        p0 = p[:, 0:1]
        p1 = p[:, 1:2]
        rr = rfull[:, base:base + 1]
        hd = (rr > p0).astype(jnp.int32)
        col = _POS2COL.get(base)
        if col is None:
            x = hd
        else:
            fv = ifull[:, col:col + 1]
            cond = jnp.abs(p0 - 0.5) > _THRESH
            x = jnp.where(cond, hd, fv)
        if const:
            p0 = jnp.broadcast_to(p0, (_BT, 1))
            p1 = jnp.broadcast_to(p1, (_BT, 1))
        return x, p0, p1

    def encode(ef, base, n, const, need_x=True):
        if n == 1:
            x, p0, p1 = leaf(ef if const else ef[0], base, const)
            return x, [x], p0, p1
        m = n // 2
        if const:
            t2 = jnp.concatenate([ef, ef], axis=1)
            u1est = checknode(t2)
            b0 = bitnode(jnp.concatenate([t2, emb0], axis=1))
            b1 = bitnode(jnp.concatenate([t2, emb1], axis=1))
        else:
            nc = len(ef)
            if nc == 1:
                half = ef[0].shape[0] // 2
                t2ch = [jnp.concatenate([ef[0][:half], ef[0][half:]], axis=1)]
            else:
                t2ch = [jnp.concatenate([ef[j], ef[j + nc // 2]], axis=1)
                        for j in range(nc // 2)]
            u1est = [checknode(t) for t in t2ch]
            spec = m * _BT <= _SPEC_ROWS
            if spec:
                b0ch, b1ch = [], []
                for t2c in t2ch:
                    rows = t2c.shape[0]
                    e0 = jnp.broadcast_to(emb0, (rows, _D))
                    e1 = jnp.broadcast_to(emb1, (rows, _D))
                    b0ch.append(bitnode(jnp.concatenate([t2c, e0], axis=1)))
                    b1ch.append(bitnode(jnp.concatenate([t2c, e1], axis=1)))

        u1, x1ch, pl0, pl1 = encode(u1est, base, m, const)

        u2est = []
        if const:
            for x1c in x1ch:
                u2est.append(jnp.where(x1c == 1, b1, b0))
        elif spec:
            for j in range(len(t2ch)):
                x1c = x1ch[j] if len(x1ch) > 1 else x1ch[0]
                u2est.append(jnp.where(x1c == 1, b1ch[j], b0ch[j]))
        else:
            for j, t2c in enumerate(t2ch):
                x1c = x1ch[j] if len(x1ch) > 1 else x1ch[0]
                u2est.append(bitnode(
                    jnp.concatenate([t2c, ue_of(x1c)], axis=1)))
        u2est = _norm(u2est)

        u2, x2ch, pr0, pr1 = encode(u2est, base + m, m, False)

        u = jnp.concatenate([u1, u2], axis=1)
        p0 = jnp.concatenate([pl0, pr0], axis=1)
        p1 = jnp.concatenate([pl1, pr1], axis=1)
        if not need_x:
            return u, None, p0, p1
        vch = [jnp.bitwise_xor(a, b) for a, b in zip(x1ch, x2ch)]
        xch = _norm(vch + x2ch)
        return u, xch, p0, p1

    def spine():
        e_row = emb_obs_ref[2:3, :]
        t2_root = jnp.concatenate([e_row, e_row], axis=1)
        ef_8l = checknode(t2_root)
        t2_8l = jnp.concatenate([ef_8l, ef_8l], axis=1)
        ef_q1 = checknode(t2_8l)
        return t2_root, t2_8l, ef_q1

    return checknode, bitnode, ue_of, bit_both, encode, spine


_STD = 14


def _q1_body(*refs):
    (uq_ref, p0_ref, p1_ref, xb_ref) = refs[_STD:]
    _ck, _bn, _ue, _bb, encode, spine = _common(*refs[:_STD])
    _t2r, _t28, ef_q1 = spine()
    u, xch, p0, p1 = encode(ef_q1, 0, _NQ, True)
    uq_ref[...] = u
    p0_ref[...] = p0
    p1_ref[...] = p1
    for j, c in enumerate(_norm(xch)):
        xb_ref[0, 0:c.shape[0], j:j + 1] = c


def _q2_body(*refs):
    xq1_ref = refs[_STD]
    (uq_ref, p0_ref, p1_ref, xb_ref) = refs[_STD + 1:]
    _ck, _bn, _ue, bit_both, encode, spine = _common(*refs[:_STD])
    _t2r, t2_8l, _efq1 = spine()
    b0, b1 = bit_both(t2_8l)
    u2est = [jnp.where(xq1_ref[0, :, j:j + 1] == 1, b1, b0)
             for j in range(_QCH)]
    u, x2ch, p0, p1 = encode(_norm(u2est), _NQ, _NQ, False)
    uq_ref[...] = u
    p0_ref[...] = p0
    p1_ref[...] = p1
    for j in range(_QCH):
        v = jnp.bitwise_xor(xq1_ref[0, :, j:j + 1], x2ch[j])
        xb_ref[0, :, j:j + 1] = v
    for j in range(_QCH):
        xb_ref[0, :, _QCH + j: _QCH + j + 1] = x2ch[j]


def _q3_body(*refs):
    x8l_ref = refs[_STD]
    (uq_ref, p0_ref, p1_ref, xb_ref, t2b_ref) = refs[_STD + 1:]
    checknode, _bn, _ue, bit_both, encode, spine = _common(*refs[:_STD])
    t2_root, _t28, _efq1 = spine()
    b0, b1 = bit_both(t2_root)
    ef8r = [jnp.where(x8l_ref[0, :, j:j + 1] == 1, b1, b0)
            for j in range(_HCH)]
    t2ch = [jnp.concatenate([ef8r[j], ef8r[j + _HCH // 2]], axis=1)
            for j in range(_HCH // 2)]
    for j, t in enumerate(t2ch):
        t2b_ref[0, :, 32 * j: 32 * (j + 1)] = t
    u1est = [checknode(t) for t in t2ch]
    u, xch, p0, p1 = encode(u1est, 2 * _NQ, _NQ, False)
    uq_ref[...] = u
    p0_ref[...] = p0
    p1_ref[...] = p1
    for j, c in enumerate(_norm(xch)):
        xb_ref[0, 0:c.shape[0], j:j + 1] = c


def _q4_body(*refs):
    g_ref, t2b_ref, xq3_ref, u1_ref, u2_ref, u3_ref = refs[_STD:_STD + 6]
    (x_ref, uq_ref, p0_ref, p1_ref) = refs[_STD + 6:]
    _ck, bitnode, ue_of, _bb, encode, _spine = _common(*refs[:_STD])
    u2est = []
    for j in range(_QCH):
        t2c = t2b_ref[0, :, 32 * j: 32 * (j + 1)]
        x1c = xq3_ref[0, :, j:j + 1]
        u2est.append(bitnode(jnp.concatenate([t2c, ue_of(x1c)], axis=1)))
    u, _x, p0, p1 = encode(_norm(u2est), 3 * _NQ, _NQ, False, need_x=False)
    uq_ref[...] = u
    p0_ref[...] = p0
    p1_ref[...] = p1
    ufull = jnp.concatenate([u1_ref[...], u2_ref[...], u3_ref[...], u],
                            axis=1)
    xf = jnp.dot(ufull.astype(jnp.float32), g_ref[...])
    x_ref[...] = jnp.bitwise_and(xf.astype(jnp.int32), 1)


def _full(shape):
    return pl.BlockSpec(shape, lambda: (0,) * len(shape))


_STD_SPECS = [
    _full((_BT, _K)), _full((_BT, _N)),
    _full((3, _D)), _full((3, _D)),
    _full((2 * _D, _H)), _full((_H,)), _full((_H, _D)), _full((_D,)),
    _full((3 * _D, _H)), _full((_H,)), _full((_H, _D)), _full((_D,)),
    _full((_D, 2)), _full((2,)),
]

_QOUT = [
    jax.ShapeDtypeStruct((_BT, _NQ), jnp.int32),
    jax.ShapeDtypeStruct((_BT, _NQ), jnp.float32),
    jax.ShapeDtypeStruct((_BT, _NQ), jnp.float32),
]
_QOUT_SPECS = [_full((_BT, _NQ)), _full((_BT, _NQ)), _full((_BT, _NQ))]


def _bits(nch):
    return (jax.ShapeDtypeStruct((1, _CHUNK, nch), jnp.int32),
            _full((1, _CHUNK, nch)))


def kernel(info_bits, r, emb_obs, emb_lab, W_c1, b_c1, W_c2, b_c2,
           W_b1, b_b1, W_b2, b_b2, W_llr, b_llr):
    std = (info_bits, r, emb_obs, emb_lab, W_c1, b_c1, W_c2, b_c2,
           W_b1, b_b1, W_b2, b_b2, W_llr, b_llr)
    xq1_t, xq1_s = _bits(_QCH)
    x8l_t, x8l_s = _bits(_HCH)
    xq3_t, xq3_s = _bits(_QCH)
    t2b_t = jax.ShapeDtypeStruct((1, _CHUNK, 32 * _QCH), jnp.float32)
    t2b_s = _full((1, _CHUNK, 32 * _QCH))

    u1, p01, p11, xq1 = pl.pallas_call(
        _q1_body, in_specs=list(_STD_SPECS),
        out_specs=_QOUT_SPECS + [xq1_s],
        out_shape=_QOUT + [xq1_t],
    )(*std)

    u2, p02, p12, x8l = pl.pallas_call(
        _q2_body, in_specs=list(_STD_SPECS) + [xq1_s],
        out_specs=_QOUT_SPECS + [x8l_s],
        out_shape=_QOUT + [x8l_t],
    )(*std, xq1)

    u3, p03, p13, xq3, t2b = pl.pallas_call(
        _q3_body, in_specs=list(_STD_SPECS) + [x8l_s],
        out_specs=_QOUT_SPECS + [xq3_s, t2b_s],
        out_shape=_QOUT + [xq3_t, t2b_t],
    )(*std, x8l)

    x2d, u4, p04, p14 = pl.pallas_call(
        _q4_body,
        in_specs=list(_STD_SPECS) + [_full((_N, _N)), t2b_s, xq3_s,
                                     _full((_BT, _NQ)), _full((_BT, _NQ)),
                                     _full((_BT, _NQ))],
        out_specs=[_full((_BT, _N))] + _QOUT_SPECS,
        out_shape=[jax.ShapeDtypeStruct((_BT, _N), jnp.int32)] + _QOUT,
    )(*std, jnp.asarray(_G), t2b, xq3, u1, u2, u3)

    x = x2d[..., None]
    u = jnp.concatenate([u1, u2, u3, u4], axis=1)[..., None]
    p0 = jnp.concatenate([p01, p02, p03, p04], axis=1)
    p1 = jnp.concatenate([p11, p12, p13, p14], axis=1)
    p_u = jnp.concatenate([p0[..., None], p1[..., None]], axis=2)
    fmask = np.ones((_N, 1), dtype=np.int32)
    fmask[_INFO_SET, 0] = 2
    f = jnp.broadcast_to(jnp.asarray(fmask)[None], (_B, _N, 1))
    return x, f, u, p_u, r
